# baseline (speedup 1.0000x reference)
import functools

import jax
import jax.numpy as jnp
from jax import lax
from jax.experimental import pallas as pl
from jax.experimental.pallas import tpu as pltpu
from jax.experimental.pallas import tpu_sc as plsc

_NB = 4
_NV = 10000
_DY = 128
_DE = 64
_NW = 32
_ROWS = _NB * _NV
_RPW = _ROWS // _NW
_NCHUNK = 10
_C = _RPW // _NCHUNK
_NSLOT = 3


@functools.lru_cache(maxsize=1)
def _build():
    mesh = plsc.VectorSubcoreMesh(core_axis_name="c", subcore_axis_name="s")

    @functools.partial(
        pl.kernel,
        out_type=(
            jax.ShapeDtypeStruct((_ROWS, _DY), jnp.float32),
            jax.ShapeDtypeStruct((_ROWS, _DE), jnp.int32),
            jax.ShapeDtypeStruct((_ROWS, _DE), jnp.float32),
        ),
        mesh=mesh,
        compiler_params=pltpu.CompilerParams(
            use_tc_tiling_on_sc=False, needs_layout_passes=False
        ),
        scratch_types=[
            pltpu.VMEM((_NCHUNK, _C), jnp.int32),
            pltpu.VMEM((_NV,), jnp.int32),
            pltpu.VMEM((_NSLOT, _C, _DY), jnp.float32),
            pltpu.VMEM((_NSLOT, _C, _DE), jnp.int32),
            pltpu.VMEM((_NSLOT, _C, _DE), jnp.float32),
            [pltpu.SemaphoreType.DMA] * _NSLOT,
            [pltpu.SemaphoreType.DMA] * _NSLOT,
            [pltpu.SemaphoreType.DMA] * _NSLOT,
            [pltpu.SemaphoreType.DMA] * _NSLOT,
            [pltpu.SemaphoreType.DMA] * _NSLOT,
            [pltpu.SemaphoreType.DMA] * _NSLOT,
        ],
    )
    def _shuffle(
        y_hbm, e_hbm, f_hbm, idx_hbm, s_hbm,
        y_out, e_out, f_out,
        idx_v, s_v, ybuf, ebuf, fbuf,
        gsy, ssy, gse, sse, gsf, ssf,
    ):
        wid = lax.axis_index("s") * 2 + lax.axis_index("c")
        row0 = wid * _RPW
        pltpu.sync_copy(idx_hbm.at[wid], idx_v)
        pltpu.sync_copy(s_hbm, s_v)

        def _gather(c):
            k = c % _NSLOT
            ix = idx_v.at[c]
            return (
                pltpu.async_copy(y_hbm.at[ix], ybuf.at[k], gsy[k]),
                pltpu.async_copy(e_hbm.at[ix], ebuf.at[k], gse[k]),
                pltpu.async_copy(f_hbm.at[ix], fbuf.at[k], gsf[k]),
            )

        def _scatter(c):
            k = c % _NSLOT
            dst = pl.ds(row0 + c * _C, _C)
            return (
                pltpu.async_copy(ybuf.at[k], y_out.at[dst], ssy[k]),
                pltpu.async_copy(ebuf.at[k], e_out.at[dst], sse[k]),
                pltpu.async_copy(fbuf.at[k], f_out.at[dst], ssf[k]),
            )

        g = [None] * _NCHUNK
        sc = [None] * _NCHUNK
        g[0] = _gather(0)
        g[1] = _gather(1)
        for c in range(_NCHUNK):
            k = c % _NSLOT
            g[c][1].wait()

            def _remap(r, _):
                for q in range(_DE // 16):
                    col = pl.ds(q * 16, 16)
                    ev = ebuf[k, r, col]
                    ebuf[k, r, col] = plsc.load_gather(s_v, [ev])
                return 0

            lax.fori_loop(0, _C, _remap, 0)
            g[c][0].wait()
            g[c][2].wait()
            sc[c] = _scatter(c)
            if c + 2 < _NCHUNK:
                if c >= 1:
                    for d in sc[c - 1]:
                        d.wait()
                g[c + 2] = _gather(c + 2)
        for c in (_NCHUNK - 3, _NCHUNK - 2, _NCHUNK - 1):
            for d in sc[c]:
                d.wait()

    return _shuffle


def _stage_s():
    return jax.random.permutation(
        jax.random.key(42), jnp.arange(_NV, dtype=jnp.int32)
    )


def kernel(y, e, f):
    s = _stage_s()
    idx = (jnp.arange(_NB, dtype=jnp.int32)[:, None] * _NV + s[None, :]).reshape(
        _NW, _NCHUNK, _C
    )
    y2, e2, f2 = _build()(
        y.reshape(_ROWS, _DY),
        e.reshape(_ROWS, _DE),
        f.reshape(_ROWS, _DE),
        idx,
        s,
    )
    return (
        y2.reshape(_NB, _NV, _DY),
        e2.reshape(_NB, _NV, 4, 16),
        f2.reshape(_NB, _NV, 4, 16),
        s,
        s,
    )
